# baseline (speedup 1.0000x reference)
import functools
import math

import jax
import jax.numpy as jnp
import numpy as np
from jax import lax
from jax.experimental import pallas as pl
from jax.experimental.pallas import tpu as pltpu
from jax.experimental.pallas import tpu_sc as plsc

_H = 224
_W = 224
_NS = 16
_SR = 16
_NSA = _H // _SR
_PW = 240
_NC = _W // 16


def _bf(v):
    return float(np.asarray(v, dtype=jnp.bfloat16).astype(np.float32))


def _gauss2d():
    v = [math.exp(-(i * i) / 2.0) for i in (-2, -1, 0, 1, 2)]
    g = np.outer(np.asarray(v, np.float32), np.asarray(v, np.float32))
    g = (g / g.sum()).astype(np.float32)
    return [[_bf(g[i, j]) for j in range(5)] for i in range(5)]


_GK = _gauss2d()


def _sh(a, d):
    if d == 0:
        return a
    z = jnp.zeros((abs(d), a.shape[1]), a.dtype)
    if d > 0:
        return jnp.concatenate([a[d:], z], axis=0)
    return jnp.concatenate([z, a[:d]], axis=0)


def _sw(a, d):
    if d == 0:
        return a
    z = jnp.zeros((a.shape[0], abs(d)), a.dtype)
    if d > 0:
        return jnp.concatenate([a[:, d:], z], axis=1)
    return jnp.concatenate([z, a[:, :d]], axis=1)


def _conv5_bf16(a):
    ab = a.astype(jnp.bfloat16).astype(jnp.float32)
    shs = [_sh(ab, k) for k in (-2, -1, 0, 1, 2)]
    acc = None
    for j in range(5):
        cj = _GK[0][j] * shs[0]
        for i in range(1, 5):
            cj = cj + _GK[i][j] * shs[i]
        t = _sw(cj, j - 2)
        acc = t if acc is None else acc + t
    return acc


def _sobel_bf16(a):
    ab = a.astype(jnp.bfloat16).astype(jnp.float32)
    sm = _sh(ab, -1)
    sp = _sh(ab, 1)
    t1 = sm + 2.0 * ab + sp
    gx = _sw(t1, 1) - _sw(t1, -1)
    t2 = sp - sm
    gy = _sw(t2, -1) + 2.0 * t2 + _sw(t2, 1)
    return gx, gy


def _embed(a):
    return jnp.concatenate(
        [jnp.zeros((_H, 1), a.dtype), a,
         jnp.zeros((_H, _PW - _W - 1), a.dtype)], axis=1)


def _dense_body(x_ref, e0_ref, low_ref):
    x0 = x_ref[0, 0]
    x1 = x_ref[0, 1]
    x2 = x_ref[0, 2]
    g = 0.299 * x0 + 0.587 * x1 + 0.114 * x2

    s = _conv5_bf16(_conv5_bf16(g))
    gx, gy = _sobel_bf16(s)

    mag = jnp.sqrt(gx * gx + gy * gy)

    ax = jnp.abs(gx)
    ay = jnp.abs(gy)
    c0 = ay <= 0.41421356237309503 * ax
    c90 = ay >= 2.414213562373095 * ax
    c45 = jnp.logical_and(jnp.logical_not(c0), jnp.logical_not(c90))
    c45 = jnp.logical_and(c45, gx * gy > 0)

    swm = _sw(mag, -1)
    swp = _sw(mag, 1)
    u = _sh(mag, -1)
    d = _sh(mag, 1)
    ul = _sh(swm, -1)
    dl = _sh(swm, 1)
    ur = _sh(swp, -1)
    dr = _sh(swp, 1)

    nmax = jnp.where(
        c0, jnp.maximum(swm, swp),
        jnp.where(c45, jnp.maximum(ur, dl),
                  jnp.where(c90, jnp.maximum(u, d), jnp.maximum(ul, dr))))

    iy = lax.broadcasted_iota(jnp.int32, (_H, _W), 0)
    ix = lax.broadcasted_iota(jnp.int32, (_H, _W), 1)
    interior = jnp.logical_and(
        jnp.logical_and(iy >= 1, iy <= _H - 2),
        jnp.logical_and(ix >= 1, ix <= _W - 2))

    keep = jnp.logical_and(interior, mag >= nmax)
    nms = jnp.where(keep, mag, 0.0)
    lowm = jnp.where(nms > 0.1, 1.0, 0.0)
    edge0 = jnp.where(nms > 0.2, 1.0, 0.0)

    e0_ref[0] = _embed(edge0)
    low_ref[0] = _embed(lowm)


def _dense_call(x):
    return pl.pallas_call(
        _dense_body,
        grid=(2,),
        in_specs=[pl.BlockSpec((1, 3, _H, _W), lambda b: (b, 0, 0, 0))],
        out_specs=(pl.BlockSpec((1, _H, _PW), lambda b: (b, 0, 0)),
                   pl.BlockSpec((1, _H, _PW), lambda b: (b, 0, 0))),
        out_shape=(jax.ShapeDtypeStruct((2, _H, _PW), jnp.float32),
                   jax.ShapeDtypeStruct((2, _H, _PW), jnp.float32)),
    )(x)


@functools.cache
def _sc_hyst_call():
    mesh = plsc.VectorSubcoreMesh(core_axis_name="c", subcore_axis_name="s")

    @functools.partial(
        pl.kernel,
        out_type=jax.ShapeDtypeStruct((2, 1, _H, _W), jnp.float32),
        mesh=mesh,
        scratch_types=[
            pltpu.VMEM((_SR + 2, _PW), jnp.float32),
            pltpu.VMEM((_SR, _PW), jnp.float32),
            pltpu.VMEM((_SR, _W), jnp.float32),
            pltpu.VMEM((16,), jnp.float32),
            pltpu.VMEM((_NS, 16), jnp.float32),
            pltpu.VMEM_SHARED((_NS, 2, _PW), jnp.float32),
            pltpu.VMEM_SHARED((_NS, 16), jnp.float32),
        ],
        compiler_params=pltpu.CompilerParams(
            use_tc_tiling_on_sc=False, needs_layout_passes=False),
    )
    def hyst(e0_hbm, low_hbm, out_hbm, ebuf, lowbuf, obuf, flagout, flagin,
             halo_sh, chg_sh):
        cid = lax.axis_index("c")
        sid = lax.axis_index("s")
        row0 = sid * _SR
        active = sid < _NSA

        iota = lax.iota(jnp.int32, 16)
        neg1 = jnp.full((16,), -1, jnp.int32)
        one = jnp.full((16,), 1.0, jnp.float32)
        zero = jnp.zeros((16,), jnp.float32)

        for r in range(_SR + 2):
            for c in range(_PW // 16):
                ebuf[r, pl.ds(16 * c, 16)] = zero
        for r in range(_SR):
            for c in range(_PW // 16):
                lowbuf[r, pl.ds(16 * c, 16)] = zero

        @pl.when(active)
        def _():
            pltpu.sync_copy(e0_hbm.at[cid, pl.ds(row0, _SR), :],
                            ebuf.at[pl.ds(1, _SR), :])
            pltpu.sync_copy(low_hbm.at[cid, pl.ds(row0, _SR), :], lowbuf)

        def exchange():
            pltpu.sync_copy(ebuf.at[1, :], halo_sh.at[sid, 0, :])
            pltpu.sync_copy(ebuf.at[_SR, :], halo_sh.at[sid, 1, :])
            plsc.subcore_barrier()

            @pl.when(sid > 0)
            def _():
                pltpu.sync_copy(halo_sh.at[sid - 1, 1, :], ebuf.at[0, :])

            @pl.when(sid < _NS - 1)
            def _():
                pltpu.sync_copy(halo_sh.at[sid + 1, 0, :],
                                ebuf.at[_SR + 1, :])

            plsc.subcore_barrier()

        def nbr8(r, b):
            m = jnp.maximum(ebuf[r - 1, pl.ds(b - 1, 16)],
                            ebuf[r - 1, pl.ds(b, 16)])
            m = jnp.maximum(m, ebuf[r - 1, pl.ds(b + 1, 16)])
            m = jnp.maximum(m, ebuf[r, pl.ds(b - 1, 16)])
            m = jnp.maximum(m, ebuf[r, pl.ds(b + 1, 16)])
            m = jnp.maximum(m, ebuf[r + 1, pl.ds(b - 1, 16)])
            m = jnp.maximum(m, ebuf[r + 1, pl.ds(b, 16)])
            m = jnp.maximum(m, ebuf[r + 1, pl.ds(b + 1, 16)])
            return m

        def down_row(r, ch):
            mcar = jnp.int32(-1)
            kcar = jnp.int32(-1)
            for c in range(_NC):
                b = 1 + 16 * c
                ctr = ebuf[r, pl.ds(b, 16)]
                lw = lowbuf[r - 1, pl.ds(b, 16)]
                nb = nbr8(r, b)
                islow = lw > 0.0
                v0 = jnp.where(jnp.logical_and(islow, nb > 0.0), one, ctr)
                pos = iota + (16 * c)
                av = jnp.where(v0 > 0.0, pos, neg1)
                bv = jnp.where(islow, neg1, pos)
                mm = jnp.maximum(plsc.cummax(av), mcar)
                kk = jnp.maximum(plsc.cummax(bv), kcar)
                nv = jnp.where(jnp.logical_and(islow, mm > kk), one, v0)
                ebuf[r, pl.ds(b, 16)] = nv
                ch = jnp.maximum(ch, nv - ctr)
                mcar = jnp.max(mm)
                kcar = jnp.max(kk)
            return ch

        def up_row(k2, ch):
            r = _SR - k2
            mcar = jnp.int32(-1)
            kcar = jnp.int32(-1)
            for cc in range(_NC):
                c = _NC - 1 - cc
                b = 1 + 16 * c
                ctr = ebuf[r, pl.ds(b, 16)]
                lw = lowbuf[r - 1, pl.ds(b, 16)]
                nb = nbr8(r, b)
                islow = lw > 0.0
                v0 = jnp.where(jnp.logical_and(islow, nb > 0.0), one, ctr)
                vr = lax.rev(v0, (0,))
                lwr = lax.rev(lw, (0,))
                islowr = lwr > 0.0
                pos = iota + (16 * cc)
                av = jnp.where(vr > 0.0, pos, neg1)
                bv = jnp.where(islowr, neg1, pos)
                mm = jnp.maximum(plsc.cummax(av), mcar)
                kk = jnp.maximum(plsc.cummax(bv), kcar)
                nvr = jnp.where(jnp.logical_and(islowr, mm > kk), one, vr)
                nv = lax.rev(nvr, (0,))
                ebuf[r, pl.ds(b, 16)] = nv
                ch = jnp.maximum(ch, nv - ctr)
                mcar = jnp.max(mm)
                kcar = jnp.max(kk)
            return ch

        def round_body(_):
            ch = jnp.zeros((16,), jnp.float32)
            ch = lax.fori_loop(1, _SR + 1, down_row, ch)
            ch = lax.fori_loop(0, _SR, up_row, ch)
            pltpu.sync_copy(ebuf.at[1, :], halo_sh.at[sid, 0, :])
            pltpu.sync_copy(ebuf.at[_SR, :], halo_sh.at[sid, 1, :])
            flagout[...] = jnp.full((16,), jnp.max(ch), jnp.float32)
            pltpu.sync_copy(flagout, chg_sh.at[sid, :])
            plsc.subcore_barrier()

            @pl.when(sid > 0)
            def _():
                pltpu.sync_copy(halo_sh.at[sid - 1, 1, :], ebuf.at[0, :])

            @pl.when(sid < _NS - 1)
            def _():
                pltpu.sync_copy(halo_sh.at[sid + 1, 0, :],
                                ebuf.at[_SR + 1, :])

            pltpu.sync_copy(chg_sh, flagin)
            m = flagin[0, :]
            for i in range(1, _NS):
                m = jnp.maximum(m, flagin[i, :])
            plsc.subcore_barrier()
            return jnp.max(m)

        exchange()
        lax.while_loop(lambda c: c > 0.0, round_body, jnp.float32(1.0))

        for r in range(1, _SR + 1):
            for c in range(_NC):
                v = ebuf[r, pl.ds(1 + 16 * c, 16)]
                obuf[r - 1, pl.ds(16 * c, 16)] = 1.0 - v

        @pl.when(active)
        def _():
            pltpu.sync_copy(obuf, out_hbm.at[cid, 0, pl.ds(row0, _SR), :])

    return hyst


@jax.jit
def kernel(x):
    e0, low = _dense_call(x)
    return _sc_hyst_call()(e0, low)


# speedup vs baseline: 1.3231x; 1.3231x over previous
import functools
import math

import jax
import jax.numpy as jnp
import numpy as np
from jax import lax
from jax.experimental import pallas as pl
from jax.experimental.pallas import tpu as pltpu
from jax.experimental.pallas import tpu_sc as plsc

_H = 224
_W = 224
_NS = 16
_SR = 16
_NSA = _H // _SR
_PW = 240
_NC = _W // 16


def _bf(v):
    return float(np.asarray(v, dtype=jnp.bfloat16).astype(np.float32))


def _gauss2d():
    v = [math.exp(-(i * i) / 2.0) for i in (-2, -1, 0, 1, 2)]
    g = np.outer(np.asarray(v, np.float32), np.asarray(v, np.float32))
    g = (g / g.sum()).astype(np.float32)
    return [[_bf(g[i, j]) for j in range(5)] for i in range(5)]


_GK = _gauss2d()


def _sh(a, d):
    if d == 0:
        return a
    z = jnp.zeros((abs(d), a.shape[1]), a.dtype)
    if d > 0:
        return jnp.concatenate([a[d:], z], axis=0)
    return jnp.concatenate([z, a[:d]], axis=0)


def _sw(a, d):
    if d == 0:
        return a
    z = jnp.zeros((a.shape[0], abs(d)), a.dtype)
    if d > 0:
        return jnp.concatenate([a[:, d:], z], axis=1)
    return jnp.concatenate([z, a[:, :d]], axis=1)


def _conv5_bf16(a):
    ab = a.astype(jnp.bfloat16).astype(jnp.float32)
    shs = [_sh(ab, k) for k in (-2, -1, 0, 1, 2)]
    acc = None
    for j in range(5):
        cj = _GK[0][j] * shs[0]
        for i in range(1, 5):
            cj = cj + _GK[i][j] * shs[i]
        t = _sw(cj, j - 2)
        acc = t if acc is None else acc + t
    return acc


def _sobel_bf16(a):
    ab = a.astype(jnp.bfloat16).astype(jnp.float32)
    sm = _sh(ab, -1)
    sp = _sh(ab, 1)
    t1 = sm + 2.0 * ab + sp
    gx = _sw(t1, 1) - _sw(t1, -1)
    t2 = sp - sm
    gy = _sw(t2, -1) + 2.0 * t2 + _sw(t2, 1)
    return gx, gy


def _embed(a):
    return jnp.concatenate(
        [jnp.zeros((_H, 1), a.dtype), a,
         jnp.zeros((_H, _PW - _W - 1), a.dtype)], axis=1)


def _dense_body(x_ref, e0_ref, low_ref):
    x0 = x_ref[0, 0]
    x1 = x_ref[0, 1]
    x2 = x_ref[0, 2]
    g = 0.299 * x0 + 0.587 * x1 + 0.114 * x2

    s = _conv5_bf16(_conv5_bf16(g))
    gx, gy = _sobel_bf16(s)

    mag = jnp.sqrt(gx * gx + gy * gy)

    ax = jnp.abs(gx)
    ay = jnp.abs(gy)
    c0 = ay <= 0.41421356237309503 * ax
    c90 = ay >= 2.414213562373095 * ax
    c45 = jnp.logical_and(jnp.logical_not(c0), jnp.logical_not(c90))
    c45 = jnp.logical_and(c45, gx * gy > 0)

    swm = _sw(mag, -1)
    swp = _sw(mag, 1)
    u = _sh(mag, -1)
    d = _sh(mag, 1)
    ul = _sh(swm, -1)
    dl = _sh(swm, 1)
    ur = _sh(swp, -1)
    dr = _sh(swp, 1)

    nmax = jnp.where(
        c0, jnp.maximum(swm, swp),
        jnp.where(c45, jnp.maximum(ur, dl),
                  jnp.where(c90, jnp.maximum(u, d), jnp.maximum(ul, dr))))

    iy = lax.broadcasted_iota(jnp.int32, (_H, _W), 0)
    ix = lax.broadcasted_iota(jnp.int32, (_H, _W), 1)
    interior = jnp.logical_and(
        jnp.logical_and(iy >= 1, iy <= _H - 2),
        jnp.logical_and(ix >= 1, ix <= _W - 2))

    keep = jnp.logical_and(interior, mag >= nmax)
    nms = jnp.where(keep, mag, 0.0)
    lowm = jnp.where(nms > 0.1, 1.0, 0.0)
    edge0 = jnp.where(nms > 0.2, 1.0, 0.0)

    e0_ref[0] = _embed(edge0)
    low_ref[0] = _embed(lowm)


def _dense_call(x):
    return pl.pallas_call(
        _dense_body,
        grid=(2,),
        in_specs=[pl.BlockSpec((1, 3, _H, _W), lambda b: (b, 0, 0, 0))],
        out_specs=(pl.BlockSpec((1, _H, _PW), lambda b: (b, 0, 0)),
                   pl.BlockSpec((1, _H, _PW), lambda b: (b, 0, 0))),
        out_shape=(jax.ShapeDtypeStruct((2, _H, _PW), jnp.float32),
                   jax.ShapeDtypeStruct((2, _H, _PW), jnp.float32)),
    )(x)


@functools.cache
def _sc_hyst_call():
    mesh = plsc.VectorSubcoreMesh(core_axis_name="c", subcore_axis_name="s")

    @functools.partial(
        pl.kernel,
        out_type=jax.ShapeDtypeStruct((2, 1, _H, _W), jnp.float32),
        mesh=mesh,
        scratch_types=[
            pltpu.VMEM((_SR + 2, _PW), jnp.float32),
            pltpu.VMEM((_SR, _PW), jnp.float32),
            pltpu.VMEM((_SR, _W), jnp.float32),
            pltpu.VMEM((16,), jnp.float32),
            pltpu.VMEM((_NS, 16), jnp.float32),
            pltpu.VMEM((16,), jnp.float32),
            pltpu.VMEM((2, _PW), jnp.float32),
            pltpu.VMEM_SHARED((_NS, 2, _PW), jnp.float32),
            pltpu.VMEM_SHARED((_NS, 16), jnp.float32),
        ],
        compiler_params=pltpu.CompilerParams(
            use_tc_tiling_on_sc=False, needs_layout_passes=False),
    )
    def hyst(e0_hbm, low_hbm, out_hbm, ebuf, lowbuf, obuf, flagout, flagin,
             chbuf, hprev, halo_sh, chg_sh):
        cid = lax.axis_index("c")
        sid = lax.axis_index("s")
        row0 = sid * _SR
        active = sid < _NSA

        iota = lax.iota(jnp.int32, 16)
        neg2 = jnp.full((16,), -2, jnp.int32)
        one = jnp.full((16,), 1.0, jnp.float32)
        zero = jnp.zeros((16,), jnp.float32)

        for r in range(_SR + 2):
            for c in range(_PW // 16):
                ebuf[r, pl.ds(16 * c, 16)] = zero
        for r in range(_SR):
            for c in range(_PW // 16):
                lowbuf[r, pl.ds(16 * c, 16)] = zero

        @pl.when(active)
        def _():
            pltpu.sync_copy(e0_hbm.at[cid, pl.ds(row0, _SR), :],
                            ebuf.at[pl.ds(1, _SR), :])
            pltpu.sync_copy(low_hbm.at[cid, pl.ds(row0, _SR), :], lowbuf)

        def exchange():
            pltpu.sync_copy(ebuf.at[1, :], halo_sh.at[sid, 0, :])
            pltpu.sync_copy(ebuf.at[_SR, :], halo_sh.at[sid, 1, :])
            plsc.subcore_barrier()

            @pl.when(sid > 0)
            def _():
                pltpu.sync_copy(halo_sh.at[sid - 1, 1, :], ebuf.at[0, :])

            @pl.when(sid < _NS - 1)
            def _():
                pltpu.sync_copy(halo_sh.at[sid + 1, 0, :],
                                ebuf.at[_SR + 1, :])

            plsc.subcore_barrier()

        def nbr8(r, b):
            m = jnp.maximum(ebuf[r - 1, pl.ds(b - 1, 16)],
                            ebuf[r - 1, pl.ds(b, 16)])
            m = jnp.maximum(m, ebuf[r - 1, pl.ds(b + 1, 16)])
            m = jnp.maximum(m, ebuf[r, pl.ds(b - 1, 16)])
            m = jnp.maximum(m, ebuf[r, pl.ds(b + 1, 16)])
            m = jnp.maximum(m, ebuf[r + 1, pl.ds(b - 1, 16)])
            m = jnp.maximum(m, ebuf[r + 1, pl.ds(b, 16)])
            m = jnp.maximum(m, ebuf[r + 1, pl.ds(b + 1, 16)])
            return m

        def down_row(r, ch):
            ctrs, v0s, islows, ss, mxs = [], [], [], [], []
            for c in range(_NC):
                b = 1 + 16 * c
                ctr = ebuf[r, pl.ds(b, 16)]
                lw = lowbuf[r - 1, pl.ds(b, 16)]
                nb = nbr8(r, b)
                islow = lw > 0.0
                v0 = jnp.where(jnp.logical_and(islow, nb > 0.0), one, ctr)
                pos2 = (iota + (16 * c)) * 2
                val = jnp.where(v0 > 0.0, pos2 + 1,
                                jnp.where(islow, neg2, pos2))
                ctrs.append(ctr)
                v0s.append(v0)
                islows.append(islow)
                ss.append(plsc.cummax(val))
                mxs.append(jnp.max(val))
            car = jnp.int32(-2)
            for c in range(_NC):
                b = 1 + 16 * c
                sadj = jnp.maximum(ss[c], jnp.full((16,), car, jnp.int32))
                f = jnp.logical_and(
                    islows[c], jnp.bitwise_and(sadj, 1) == 1)
                nv = jnp.where(f, one, v0s[c])
                ebuf[r, pl.ds(b, 16)] = nv
                ch = jnp.maximum(ch, nv - ctrs[c])
                car = jnp.maximum(car, mxs[c])
            return ch

        def up_row(k2, ch):
            r = _SR - k2
            ctrs, v0s, islowrs, ss, mxs = [], [], [], [], []
            for cc in range(_NC):
                c = _NC - 1 - cc
                b = 1 + 16 * c
                ctr = ebuf[r, pl.ds(b, 16)]
                lw = lowbuf[r - 1, pl.ds(b, 16)]
                nb = nbr8(r, b)
                islow = lw > 0.0
                v0 = jnp.where(jnp.logical_and(islow, nb > 0.0), one, ctr)
                vr = lax.rev(v0, (0,))
                islowr = lax.rev(lw, (0,)) > 0.0
                pos2 = (iota + (16 * cc)) * 2
                val = jnp.where(vr > 0.0, pos2 + 1,
                                jnp.where(islowr, neg2, pos2))
                ctrs.append(ctr)
                v0s.append(vr)
                islowrs.append(islowr)
                ss.append(plsc.cummax(val))
                mxs.append(jnp.max(val))
            car = jnp.int32(-2)
            for cc in range(_NC):
                c = _NC - 1 - cc
                b = 1 + 16 * c
                sadj = jnp.maximum(ss[cc], jnp.full((16,), car, jnp.int32))
                f = jnp.logical_and(
                    islowrs[cc], jnp.bitwise_and(sadj, 1) == 1)
                nvr = jnp.where(f, one, v0s[cc])
                nv = lax.rev(nvr, (0,))
                ebuf[r, pl.ds(b, 16)] = nv
                ch = jnp.maximum(ch, nv - ctrs[cc])
                car = jnp.maximum(car, mxs[cc])
            return ch

        def snap_halos():
            hd = zero
            for c in range(_PW // 16):
                nh = ebuf[0, pl.ds(16 * c, 16)]
                hd = jnp.maximum(hd, jnp.abs(nh - hprev[0, pl.ds(16 * c, 16)]))
                hprev[0, pl.ds(16 * c, 16)] = nh
                nh = ebuf[_SR + 1, pl.ds(16 * c, 16)]
                hd = jnp.maximum(hd, jnp.abs(nh - hprev[1, pl.ds(16 * c, 16)]))
                hprev[1, pl.ds(16 * c, 16)] = nh
            return hd

        def round_body(carry):
            _, dirty = carry
            chbuf[...] = zero

            @pl.when(dirty > 0.0)
            def _():
                ch = jnp.zeros((16,), jnp.float32)
                ch = lax.fori_loop(1, _SR + 1, down_row, ch)
                ch = lax.fori_loop(0, _SR, up_row, ch)
                chbuf[...] = ch

            lch = jnp.max(chbuf[...])
            pltpu.sync_copy(ebuf.at[1, :], halo_sh.at[sid, 0, :])
            pltpu.sync_copy(ebuf.at[_SR, :], halo_sh.at[sid, 1, :])
            flagout[...] = jnp.full((16,), lch, jnp.float32)
            pltpu.sync_copy(flagout, chg_sh.at[sid, :])
            plsc.subcore_barrier()

            @pl.when(sid > 0)
            def _():
                pltpu.sync_copy(halo_sh.at[sid - 1, 1, :], ebuf.at[0, :])

            @pl.when(sid < _NS - 1)
            def _():
                pltpu.sync_copy(halo_sh.at[sid + 1, 0, :],
                                ebuf.at[_SR + 1, :])

            pltpu.sync_copy(chg_sh, flagin)
            m = flagin[0, :]
            for i in range(1, _NS):
                m = jnp.maximum(m, flagin[i, :])
            hd = snap_halos()
            plsc.subcore_barrier()
            dirty_n = jnp.where(
                jnp.logical_or(lch > 0.0, jnp.max(hd) > 0.0), 1.0, 0.0)
            return jnp.max(m), dirty_n

        exchange()
        snap_halos()
        lax.while_loop(lambda c: c[0] > 0.0, round_body,
                       (jnp.float32(1.0), jnp.float32(1.0)))

        for r in range(1, _SR + 1):
            for c in range(_NC):
                v = ebuf[r, pl.ds(1 + 16 * c, 16)]
                obuf[r - 1, pl.ds(16 * c, 16)] = 1.0 - v

        @pl.when(active)
        def _():
            pltpu.sync_copy(obuf, out_hbm.at[cid, 0, pl.ds(row0, _SR), :])

    return hyst


@jax.jit
def kernel(x):
    e0, low = _dense_call(x)
    return _sc_hyst_call()(e0, low)


# speedup vs baseline: 1.4662x; 1.1081x over previous
import functools
import math

import jax
import jax.numpy as jnp
import numpy as np
from jax import lax
from jax.experimental import pallas as pl
from jax.experimental.pallas import tpu as pltpu
from jax.experimental.pallas import tpu_sc as plsc

_H = 224
_W = 224
_NS = 16
_SR = 16
_NSA = _H // _SR
_PW = 240
_NC = _W // 16


def _bf(v):
    return float(np.asarray(v, dtype=jnp.bfloat16).astype(np.float32))


def _gauss2d():
    v = [math.exp(-(i * i) / 2.0) for i in (-2, -1, 0, 1, 2)]
    g = np.outer(np.asarray(v, np.float32), np.asarray(v, np.float32))
    g = (g / g.sum()).astype(np.float32)
    return [[_bf(g[i, j]) for j in range(5)] for i in range(5)]


_GK = _gauss2d()


def _sh(a, d):
    if d == 0:
        return a
    z = jnp.zeros((abs(d), a.shape[1]), a.dtype)
    if d > 0:
        return jnp.concatenate([a[d:], z], axis=0)
    return jnp.concatenate([z, a[:d]], axis=0)


def _sw(a, d):
    if d == 0:
        return a
    z = jnp.zeros((a.shape[0], abs(d)), a.dtype)
    if d > 0:
        return jnp.concatenate([a[:, d:], z], axis=1)
    return jnp.concatenate([z, a[:, :d]], axis=1)


def _conv5_bf16(a):
    ab = a.astype(jnp.bfloat16).astype(jnp.float32)
    shs = [_sh(ab, k) for k in (-2, -1, 0, 1, 2)]
    acc = None
    for j in range(5):
        cj = _GK[0][j] * shs[0]
        for i in range(1, 5):
            cj = cj + _GK[i][j] * shs[i]
        t = _sw(cj, j - 2)
        acc = t if acc is None else acc + t
    return acc


def _sobel_bf16(a):
    ab = a.astype(jnp.bfloat16).astype(jnp.float32)
    sm = _sh(ab, -1)
    sp = _sh(ab, 1)
    t1 = sm + 2.0 * ab + sp
    gx = _sw(t1, 1) - _sw(t1, -1)
    t2 = sp - sm
    gy = _sw(t2, -1) + 2.0 * t2 + _sw(t2, 1)
    return gx, gy


def _embed(a):
    return jnp.concatenate(
        [jnp.zeros((_H, 1), a.dtype), a,
         jnp.zeros((_H, _PW - _W - 1), a.dtype)], axis=1)


def _dense_body(x_ref, e0_ref, low_ref):
    x0 = x_ref[0, 0]
    x1 = x_ref[0, 1]
    x2 = x_ref[0, 2]
    g = 0.299 * x0 + 0.587 * x1 + 0.114 * x2

    s = _conv5_bf16(_conv5_bf16(g))
    gx, gy = _sobel_bf16(s)

    mag = jnp.sqrt(gx * gx + gy * gy)

    ax = jnp.abs(gx)
    ay = jnp.abs(gy)
    c0 = ay <= 0.41421356237309503 * ax
    c90 = ay >= 2.414213562373095 * ax
    c45 = jnp.logical_and(jnp.logical_not(c0), jnp.logical_not(c90))
    c45 = jnp.logical_and(c45, gx * gy > 0)

    swm = _sw(mag, -1)
    swp = _sw(mag, 1)
    u = _sh(mag, -1)
    d = _sh(mag, 1)
    ul = _sh(swm, -1)
    dl = _sh(swm, 1)
    ur = _sh(swp, -1)
    dr = _sh(swp, 1)

    nmax = jnp.where(
        c0, jnp.maximum(swm, swp),
        jnp.where(c45, jnp.maximum(ur, dl),
                  jnp.where(c90, jnp.maximum(u, d), jnp.maximum(ul, dr))))

    iy = lax.broadcasted_iota(jnp.int32, (_H, _W), 0)
    ix = lax.broadcasted_iota(jnp.int32, (_H, _W), 1)
    interior = jnp.logical_and(
        jnp.logical_and(iy >= 1, iy <= _H - 2),
        jnp.logical_and(ix >= 1, ix <= _W - 2))

    keep = jnp.logical_and(interior, mag >= nmax)
    nms = jnp.where(keep, mag, 0.0)
    lowm = jnp.where(nms > 0.1, 1.0, 0.0)
    edge0 = jnp.where(nms > 0.2, 1.0, 0.0)

    e0_ref[0] = _embed(edge0)
    low_ref[0] = _embed(lowm)


def _dense_call(x):
    return pl.pallas_call(
        _dense_body,
        grid=(2,),
        in_specs=[pl.BlockSpec((1, 3, _H, _W), lambda b: (b, 0, 0, 0))],
        out_specs=(pl.BlockSpec((1, _H, _PW), lambda b: (b, 0, 0)),
                   pl.BlockSpec((1, _H, _PW), lambda b: (b, 0, 0))),
        out_shape=(jax.ShapeDtypeStruct((2, _H, _PW), jnp.float32),
                   jax.ShapeDtypeStruct((2, _H, _PW), jnp.float32)),
    )(x)


@functools.cache
def _sc_hyst_call():
    mesh = plsc.VectorSubcoreMesh(core_axis_name="c", subcore_axis_name="s")

    @functools.partial(
        pl.kernel,
        out_type=jax.ShapeDtypeStruct((2, 1, _H, _W), jnp.float32),
        mesh=mesh,
        scratch_types=[
            pltpu.VMEM((_SR + 2, _PW), jnp.float32),
            pltpu.VMEM((_SR, _PW), jnp.float32),
            pltpu.VMEM((_SR, _W), jnp.float32),
            pltpu.VMEM((16,), jnp.float32),
            pltpu.VMEM((_NS, 16), jnp.float32),
            pltpu.VMEM((16,), jnp.float32),
            pltpu.VMEM((2, _PW), jnp.float32),
            pltpu.VMEM_SHARED((_NS, 2, 2, _PW), jnp.float32),
            pltpu.VMEM_SHARED((_NS, 2, 16), jnp.float32),
        ],
        compiler_params=pltpu.CompilerParams(
            use_tc_tiling_on_sc=False, needs_layout_passes=False),
    )
    def hyst(e0_hbm, low_hbm, out_hbm, ebuf, lowbuf, obuf, flagout, flagin,
             chbuf, hprev, halo_sh, chg_sh):
        cid = lax.axis_index("c")
        sid = lax.axis_index("s")
        row0 = sid * _SR
        active = sid < _NSA

        iota = lax.iota(jnp.int32, 16)
        neg2 = jnp.full((16,), -2, jnp.int32)
        one = jnp.full((16,), 1.0, jnp.float32)
        zero = jnp.zeros((16,), jnp.float32)

        for r in range(_SR + 2):
            for c in range(_PW // 16):
                ebuf[r, pl.ds(16 * c, 16)] = zero
        for r in range(_SR):
            for c in range(_PW // 16):
                lowbuf[r, pl.ds(16 * c, 16)] = zero

        @pl.when(active)
        def _():
            pltpu.sync_copy(e0_hbm.at[cid, pl.ds(row0, _SR), :],
                            ebuf.at[pl.ds(1, _SR), :])
            pltpu.sync_copy(low_hbm.at[cid, pl.ds(row0, _SR), :], lowbuf)

        def exchange():
            pltpu.sync_copy(ebuf.at[1, :], halo_sh.at[sid, 1, 0, :])
            pltpu.sync_copy(ebuf.at[_SR, :], halo_sh.at[sid, 1, 1, :])
            plsc.subcore_barrier()

            @pl.when(sid > 0)
            def _():
                pltpu.sync_copy(halo_sh.at[sid - 1, 1, 1, :], ebuf.at[0, :])

            @pl.when(sid < _NS - 1)
            def _():
                pltpu.sync_copy(halo_sh.at[sid + 1, 1, 0, :],
                                ebuf.at[_SR + 1, :])

            plsc.subcore_barrier()

        def nbr6(r, b):
            m = jnp.maximum(ebuf[r - 1, pl.ds(b - 1, 16)],
                            ebuf[r - 1, pl.ds(b, 16)])
            m = jnp.maximum(m, ebuf[r - 1, pl.ds(b + 1, 16)])
            m = jnp.maximum(m, ebuf[r + 1, pl.ds(b - 1, 16)])
            m = jnp.maximum(m, ebuf[r + 1, pl.ds(b, 16)])
            m = jnp.maximum(m, ebuf[r + 1, pl.ds(b + 1, 16)])
            return m

        _HALF = _NC // 2

        def down_row(r, ch):
            car = jnp.int32(-2)
            for half in range(2):
                chunks = range(_HALF * half, _HALF * (half + 1))
                v0s, islows, ss, mxs = [], [], [], []
                for c in chunks:
                    b = 1 + 16 * c
                    ctr = ebuf[r, pl.ds(b, 16)]
                    lw = lowbuf[r - 1, pl.ds(b, 16)]
                    islow = lw > 0.0
                    v0 = jnp.where(
                        jnp.logical_and(islow, nbr6(r, b) > 0.0), one, ctr)
                    ch = jnp.maximum(ch, v0 - ctr)
                    pos2 = (iota + (16 * c)) * 2
                    val = jnp.where(v0 > 0.0, pos2 + 1,
                                    jnp.where(islow, neg2, pos2))
                    v0s.append(v0)
                    islows.append(islow)
                    ss.append(plsc.cummax(val))
                    mxs.append(jnp.max(val))
                for i, c in enumerate(chunks):
                    b = 1 + 16 * c
                    sadj = jnp.maximum(ss[i], jnp.full((16,), car, jnp.int32))
                    f = jnp.logical_and(
                        islows[i], jnp.bitwise_and(sadj, 1) == 1)
                    nv = jnp.where(f, one, v0s[i])
                    ebuf[r, pl.ds(b, 16)] = nv
                    ch = jnp.maximum(ch, nv - v0s[i])
                    car = jnp.maximum(car, mxs[i])
            return ch

        def up_row(k2, ch):
            r = _SR - k2
            car = jnp.int32(-2)
            for half in range(2):
                chunkccs = range(_HALF * half, _HALF * (half + 1))
                v0s, islowrs, ss, mxs = [], [], [], []
                for cc in chunkccs:
                    c = _NC - 1 - cc
                    b = 1 + 16 * c
                    ctr = ebuf[r, pl.ds(b, 16)]
                    lw = lowbuf[r - 1, pl.ds(b, 16)]
                    islow = lw > 0.0
                    v0 = jnp.where(
                        jnp.logical_and(islow, nbr6(r, b) > 0.0), one, ctr)
                    ch = jnp.maximum(ch, v0 - ctr)
                    vr = lax.rev(v0, (0,))
                    islowr = lax.rev(lw, (0,)) > 0.0
                    pos2 = (iota + (16 * cc)) * 2
                    val = jnp.where(vr > 0.0, pos2 + 1,
                                    jnp.where(islowr, neg2, pos2))
                    v0s.append(vr)
                    islowrs.append(islowr)
                    ss.append(plsc.cummax(val))
                    mxs.append(jnp.max(val))
                for i, cc in enumerate(chunkccs):
                    c = _NC - 1 - cc
                    b = 1 + 16 * c
                    sadj = jnp.maximum(ss[i], jnp.full((16,), car, jnp.int32))
                    f = jnp.logical_and(
                        islowrs[i], jnp.bitwise_and(sadj, 1) == 1)
                    nvr = jnp.where(f, one, v0s[i])
                    nv = lax.rev(nvr, (0,))
                    ebuf[r, pl.ds(b, 16)] = nv
                    ch = jnp.maximum(ch, nv - lax.rev(v0s[i], (0,)))
                    car = jnp.maximum(car, mxs[i])
            return ch

        def snap_halos():
            hd = zero
            for c in range(_PW // 16):
                nh = ebuf[0, pl.ds(16 * c, 16)]
                hd = jnp.maximum(hd, jnp.abs(nh - hprev[0, pl.ds(16 * c, 16)]))
                hprev[0, pl.ds(16 * c, 16)] = nh
                nh = ebuf[_SR + 1, pl.ds(16 * c, 16)]
                hd = jnp.maximum(hd, jnp.abs(nh - hprev[1, pl.ds(16 * c, 16)]))
                hprev[1, pl.ds(16 * c, 16)] = nh
            return hd

        def round_body(carry):
            _, dirty, par = carry
            pi = par.astype(jnp.int32)
            chbuf[...] = zero

            @pl.when(dirty > 0.0)
            def _():
                ch = jnp.zeros((16,), jnp.float32)
                ch = lax.fori_loop(1, _SR + 1, down_row, ch)
                ch = lax.fori_loop(0, _SR, up_row, ch)
                chbuf[...] = ch

            lch = jnp.max(chbuf[...])
            pltpu.sync_copy(ebuf.at[1, :], halo_sh.at[sid, pi, 0, :])
            pltpu.sync_copy(ebuf.at[_SR, :], halo_sh.at[sid, pi, 1, :])
            flagout[...] = jnp.full((16,), lch, jnp.float32)
            pltpu.sync_copy(flagout, chg_sh.at[sid, pi, :])
            plsc.subcore_barrier()

            @pl.when(sid > 0)
            def _():
                pltpu.sync_copy(halo_sh.at[sid - 1, pi, 1, :], ebuf.at[0, :])

            @pl.when(sid < _NS - 1)
            def _():
                pltpu.sync_copy(halo_sh.at[sid + 1, pi, 0, :],
                                ebuf.at[_SR + 1, :])

            pltpu.sync_copy(chg_sh.at[:, pi, :], flagin)
            m = flagin[0, :]
            for i in range(1, _NS):
                m = jnp.maximum(m, flagin[i, :])
            hd = snap_halos()
            dirty_n = jnp.where(
                jnp.logical_or(lch > 0.0, jnp.max(hd) > 0.0), 1.0, 0.0)
            return jnp.max(m), dirty_n, 1.0 - par

        exchange()
        snap_halos()
        lax.while_loop(lambda c: c[0] > 0.0, round_body,
                       (jnp.float32(1.0), jnp.float32(1.0),
                        jnp.float32(0.0)))

        for r in range(1, _SR + 1):
            for c in range(_NC):
                v = ebuf[r, pl.ds(1 + 16 * c, 16)]
                obuf[r - 1, pl.ds(16 * c, 16)] = 1.0 - v

        @pl.when(active)
        def _():
            pltpu.sync_copy(obuf, out_hbm.at[cid, 0, pl.ds(row0, _SR), :])

    return hyst


@jax.jit
def kernel(x):
    e0, low = _dense_call(x)
    return _sc_hyst_call()(e0, low)


# speedup vs baseline: 1.5000x; 1.0230x over previous
import functools
import math

import jax
import jax.numpy as jnp
import numpy as np
from jax import lax
from jax.experimental import pallas as pl
from jax.experimental.pallas import tpu as pltpu
from jax.experimental.pallas import tpu_sc as plsc

_H = 224
_W = 224
_NS = 16
_SR = 16
_NSA = _H // _SR
_PW = 240
_NC = _W // 16


def _bf(v):
    return float(np.asarray(v, dtype=jnp.bfloat16).astype(np.float32))


def _gauss2d():
    v = [math.exp(-(i * i) / 2.0) for i in (-2, -1, 0, 1, 2)]
    g = np.outer(np.asarray(v, np.float32), np.asarray(v, np.float32))
    g = (g / g.sum()).astype(np.float32)
    return [[_bf(g[i, j]) for j in range(5)] for i in range(5)]


_GK = _gauss2d()


def _sh(a, d):
    if d == 0:
        return a
    z = jnp.zeros((abs(d), a.shape[1]), a.dtype)
    if d > 0:
        return jnp.concatenate([a[d:], z], axis=0)
    return jnp.concatenate([z, a[:d]], axis=0)


def _sw(a, d):
    if d == 0:
        return a
    z = jnp.zeros((a.shape[0], abs(d)), a.dtype)
    if d > 0:
        return jnp.concatenate([a[:, d:], z], axis=1)
    return jnp.concatenate([z, a[:, :d]], axis=1)


def _conv5_bf16(a):
    ab = a.astype(jnp.bfloat16).astype(jnp.float32)
    shs = [_sh(ab, k) for k in (-2, -1, 0, 1, 2)]
    acc = None
    for j in range(5):
        cj = _GK[0][j] * shs[0]
        for i in range(1, 5):
            cj = cj + _GK[i][j] * shs[i]
        t = _sw(cj, j - 2)
        acc = t if acc is None else acc + t
    return acc


def _sobel_bf16(a):
    ab = a.astype(jnp.bfloat16).astype(jnp.float32)
    sm = _sh(ab, -1)
    sp = _sh(ab, 1)
    t1 = sm + 2.0 * ab + sp
    gx = _sw(t1, 1) - _sw(t1, -1)
    t2 = sp - sm
    gy = _sw(t2, -1) + 2.0 * t2 + _sw(t2, 1)
    return gx, gy


def _embed(a):
    return jnp.concatenate(
        [jnp.zeros((_H, 1), a.dtype), a,
         jnp.zeros((_H, _PW - _W - 1), a.dtype)], axis=1)


def _dense_body(x_ref, e0_ref, low_ref):
    x0 = x_ref[0, 0]
    x1 = x_ref[0, 1]
    x2 = x_ref[0, 2]
    g = 0.299 * x0 + 0.587 * x1 + 0.114 * x2

    s = _conv5_bf16(_conv5_bf16(g))
    gx, gy = _sobel_bf16(s)

    mag = jnp.sqrt(gx * gx + gy * gy)

    ax = jnp.abs(gx)
    ay = jnp.abs(gy)
    c0 = ay <= 0.41421356237309503 * ax
    c90 = ay >= 2.414213562373095 * ax
    c45 = jnp.logical_and(jnp.logical_not(c0), jnp.logical_not(c90))
    c45 = jnp.logical_and(c45, gx * gy > 0)

    swm = _sw(mag, -1)
    swp = _sw(mag, 1)
    u = _sh(mag, -1)
    d = _sh(mag, 1)
    ul = _sh(swm, -1)
    dl = _sh(swm, 1)
    ur = _sh(swp, -1)
    dr = _sh(swp, 1)

    nmax = jnp.where(
        c0, jnp.maximum(swm, swp),
        jnp.where(c45, jnp.maximum(ur, dl),
                  jnp.where(c90, jnp.maximum(u, d), jnp.maximum(ul, dr))))

    iy = lax.broadcasted_iota(jnp.int32, (_H, _W), 0)
    ix = lax.broadcasted_iota(jnp.int32, (_H, _W), 1)
    interior = jnp.logical_and(
        jnp.logical_and(iy >= 1, iy <= _H - 2),
        jnp.logical_and(ix >= 1, ix <= _W - 2))

    keep = jnp.logical_and(interior, mag >= nmax)
    nms = jnp.where(keep, mag, 0.0)
    lowm = jnp.where(nms > 0.1, 1.0, 0.0)
    edge0 = jnp.where(nms > 0.2, 1.0, 0.0)

    e0_ref[0] = _embed(edge0)
    low_ref[0] = _embed(lowm)


def _dense_call(x):
    return pl.pallas_call(
        _dense_body,
        grid=(2,),
        in_specs=[pl.BlockSpec((1, 3, _H, _W), lambda b: (b, 0, 0, 0))],
        out_specs=(pl.BlockSpec((1, _H, _PW), lambda b: (b, 0, 0)),
                   pl.BlockSpec((1, _H, _PW), lambda b: (b, 0, 0))),
        out_shape=(jax.ShapeDtypeStruct((2, _H, _PW), jnp.float32),
                   jax.ShapeDtypeStruct((2, _H, _PW), jnp.float32)),
    )(x)


@functools.cache
def _sc_hyst_call():
    mesh = plsc.VectorSubcoreMesh(core_axis_name="c", subcore_axis_name="s")

    @functools.partial(
        pl.kernel,
        out_type=jax.ShapeDtypeStruct((2, 1, _H, _W), jnp.float32),
        mesh=mesh,
        scratch_types=[
            pltpu.VMEM((_SR + 2, _PW), jnp.float32),
            pltpu.VMEM((_SR, _PW), jnp.float32),
            pltpu.VMEM((_SR, _W), jnp.float32),
            pltpu.VMEM((16,), jnp.float32),
            pltpu.VMEM((_NS, 16), jnp.float32),
            pltpu.VMEM((16,), jnp.float32),
            pltpu.VMEM((2, _PW), jnp.float32),
            pltpu.VMEM_SHARED((_NS, 2, 2, _PW), jnp.float32),
            pltpu.VMEM_SHARED((_NS, 2, 16), jnp.float32),
        ],
        compiler_params=pltpu.CompilerParams(
            use_tc_tiling_on_sc=False, needs_layout_passes=False),
    )
    def hyst(e0_hbm, low_hbm, out_hbm, ebuf, lowbuf, obuf, flagout, flagin,
             chbuf, hprev, halo_sh, chg_sh):
        cid = lax.axis_index("c")
        sid = lax.axis_index("s")
        row0 = sid * _SR
        active = sid < _NSA

        iota = lax.iota(jnp.int32, 16)
        neg2 = jnp.full((16,), -2, jnp.int32)
        one = jnp.full((16,), 1.0, jnp.float32)
        zero = jnp.zeros((16,), jnp.float32)

        for r in (0, _SR + 1):
            for c in range(_PW // 16):
                ebuf[r, pl.ds(16 * c, 16)] = zero

        @pl.when(active)
        def _():
            pltpu.sync_copy(e0_hbm.at[cid, pl.ds(row0, _SR), :],
                            ebuf.at[pl.ds(1, _SR), :])
            pltpu.sync_copy(low_hbm.at[cid, pl.ds(row0, _SR), :], lowbuf)

        def exchange():
            @pl.when(active)
            def _():
                pltpu.sync_copy(ebuf.at[1, :], halo_sh.at[sid, 1, 0, :])
                pltpu.sync_copy(ebuf.at[_SR, :], halo_sh.at[sid, 1, 1, :])

            plsc.subcore_barrier()

            @pl.when(jnp.logical_and(sid > 0, active))
            def _():
                pltpu.sync_copy(halo_sh.at[sid - 1, 1, 1, :], ebuf.at[0, :])

            @pl.when(sid < _NSA - 1)
            def _():
                pltpu.sync_copy(halo_sh.at[sid + 1, 1, 0, :],
                                ebuf.at[_SR + 1, :])

            plsc.subcore_barrier()

        def nbr6(r, b):
            m = jnp.maximum(ebuf[r - 1, pl.ds(b - 1, 16)],
                            ebuf[r - 1, pl.ds(b, 16)])
            m = jnp.maximum(m, ebuf[r - 1, pl.ds(b + 1, 16)])
            m = jnp.maximum(m, ebuf[r + 1, pl.ds(b - 1, 16)])
            m = jnp.maximum(m, ebuf[r + 1, pl.ds(b, 16)])
            m = jnp.maximum(m, ebuf[r + 1, pl.ds(b + 1, 16)])
            return m

        _HALF = _NC // 2

        def down_row(r, ch):
            car = jnp.int32(-2)
            for half in range(2):
                chunks = range(_HALF * half, _HALF * (half + 1))
                v0s, islows, ss, mxs = [], [], [], []
                for c in chunks:
                    b = 1 + 16 * c
                    ctr = ebuf[r, pl.ds(b, 16)]
                    lw = lowbuf[r - 1, pl.ds(b, 16)]
                    islow = lw > 0.0
                    v0 = jnp.where(
                        jnp.logical_and(islow, nbr6(r, b) > 0.0), one, ctr)
                    ch = jnp.maximum(ch, v0 - ctr)
                    pos2 = (iota + (16 * c)) * 2
                    val = jnp.where(v0 > 0.0, pos2 + 1,
                                    jnp.where(islow, neg2, pos2))
                    v0s.append(v0)
                    islows.append(islow)
                    ss.append(plsc.cummax(val))
                    mxs.append(jnp.max(val))
                for i, c in enumerate(chunks):
                    b = 1 + 16 * c
                    sadj = jnp.maximum(ss[i], jnp.full((16,), car, jnp.int32))
                    f = jnp.logical_and(
                        islows[i], jnp.bitwise_and(sadj, 1) == 1)
                    nv = jnp.where(f, one, v0s[i])
                    ebuf[r, pl.ds(b, 16)] = nv
                    ch = jnp.maximum(ch, nv - v0s[i])
                    car = jnp.maximum(car, mxs[i])
            return ch

        def up_row(k2, ch):
            r = _SR - k2
            car = jnp.int32(-2)
            for half in range(2):
                chunkccs = range(_HALF * half, _HALF * (half + 1))
                v0s, islowrs, ss, mxs = [], [], [], []
                for cc in chunkccs:
                    c = _NC - 1 - cc
                    b = 1 + 16 * c
                    ctr = ebuf[r, pl.ds(b, 16)]
                    lw = lowbuf[r - 1, pl.ds(b, 16)]
                    islow = lw > 0.0
                    v0 = jnp.where(
                        jnp.logical_and(islow, nbr6(r, b) > 0.0), one, ctr)
                    ch = jnp.maximum(ch, v0 - ctr)
                    vr = lax.rev(v0, (0,))
                    islowr = lax.rev(lw, (0,)) > 0.0
                    pos2 = (iota + (16 * cc)) * 2
                    val = jnp.where(vr > 0.0, pos2 + 1,
                                    jnp.where(islowr, neg2, pos2))
                    v0s.append(vr)
                    islowrs.append(islowr)
                    ss.append(plsc.cummax(val))
                    mxs.append(jnp.max(val))
                for i, cc in enumerate(chunkccs):
                    c = _NC - 1 - cc
                    b = 1 + 16 * c
                    sadj = jnp.maximum(ss[i], jnp.full((16,), car, jnp.int32))
                    f = jnp.logical_and(
                        islowrs[i], jnp.bitwise_and(sadj, 1) == 1)
                    nvr = jnp.where(f, one, v0s[i])
                    nv = lax.rev(nvr, (0,))
                    ebuf[r, pl.ds(b, 16)] = nv
                    ch = jnp.maximum(ch, nv - lax.rev(v0s[i], (0,)))
                    car = jnp.maximum(car, mxs[i])
            return ch

        def snap_halos():
            hd = zero
            for c in range(_PW // 16):
                nh = ebuf[0, pl.ds(16 * c, 16)]
                hd = jnp.maximum(hd, jnp.abs(nh - hprev[0, pl.ds(16 * c, 16)]))
                hprev[0, pl.ds(16 * c, 16)] = nh
                nh = ebuf[_SR + 1, pl.ds(16 * c, 16)]
                hd = jnp.maximum(hd, jnp.abs(nh - hprev[1, pl.ds(16 * c, 16)]))
                hprev[1, pl.ds(16 * c, 16)] = nh
            return hd

        def round_body(carry):
            _, dirty, par = carry
            pi = par.astype(jnp.int32)
            chbuf[...] = zero

            @pl.when(jnp.logical_and(dirty > 0.0, active))
            def _():
                ch = jnp.zeros((16,), jnp.float32)
                ch = lax.fori_loop(1, _SR + 1, down_row, ch)
                ch = lax.fori_loop(0, _SR, up_row, ch)
                chbuf[...] = ch

            lch = jnp.max(chbuf[...])

            @pl.when(active)
            def _():
                pltpu.sync_copy(ebuf.at[1, :], halo_sh.at[sid, pi, 0, :])
                pltpu.sync_copy(ebuf.at[_SR, :], halo_sh.at[sid, pi, 1, :])

            flagout[...] = jnp.full((16,), lch, jnp.float32)
            pltpu.sync_copy(flagout, chg_sh.at[sid, pi, :])
            plsc.subcore_barrier()

            @pl.when(jnp.logical_and(sid > 0, active))
            def _():
                pltpu.sync_copy(halo_sh.at[sid - 1, pi, 1, :], ebuf.at[0, :])

            @pl.when(sid < _NSA - 1)
            def _():
                pltpu.sync_copy(halo_sh.at[sid + 1, pi, 0, :],
                                ebuf.at[_SR + 1, :])

            pltpu.sync_copy(chg_sh.at[:, pi, :], flagin)
            m = flagin[0, :]
            for i in range(1, _NS):
                m = jnp.maximum(m, flagin[i, :])
            hd = snap_halos()
            dirty_n = jnp.where(
                jnp.logical_or(lch > 0.0, jnp.max(hd) > 0.0), 1.0, 0.0)
            return jnp.max(m), dirty_n, 1.0 - par

        exchange()
        snap_halos()
        lax.while_loop(lambda c: c[0] > 0.0, round_body,
                       (jnp.float32(1.0), jnp.float32(1.0),
                        jnp.float32(0.0)))

        for r in range(1, _SR + 1):
            for c in range(_NC):
                v = ebuf[r, pl.ds(1 + 16 * c, 16)]
                obuf[r - 1, pl.ds(16 * c, 16)] = 1.0 - v

        @pl.when(active)
        def _():
            pltpu.sync_copy(obuf, out_hbm.at[cid, 0, pl.ds(row0, _SR), :])

    return hyst


@jax.jit
def kernel(x):
    e0, low = _dense_call(x)
    return _sc_hyst_call()(e0, low)
